# baseline (speedup 1.0000x reference)
import functools

import jax
import jax.numpy as jnp
from jax import lax
from jax.experimental import pallas as pl
from jax.experimental.pallas import tpu as pltpu
from jax.experimental.pallas import tpu_sc as plsc

K = 16
L = 16
B, M, N = 4, 2048, 8192
NC, NS = 2, 16
NW = NC * NS
NCH = N // L
NSC = NCH // L
QPB = M // NW
QU = 4


def _merge_sorted(fd, fi, d, pid):
    bd_s, bi_s = plsc.sort_key_val(d, pid)
    bd_r = lax.rev(bd_s, (0,))
    bi_r = lax.rev(bi_s, (0,))
    take = (fd < bd_r) | ((fd == bd_r) & (fi < bi_r))
    md = jnp.where(take, fd, bd_r)
    mi = jnp.where(take, fi, bi_r)
    sd, si = plsc.sort_key_val(md, mi)
    return sd, si


def _sc_body(xt_hbm, yt_hbm, zt_hbm, x2t_hbm, qx_hbm, qy_hbm, qz_hbm, c2_hbm,
             od_hbm, oi_hbm,
             xv, yv, zv, x2v, qxv, qyv, qzv, c2sv, cmv, ccv, odv, oiv):
    wid = lax.axis_index("s") * NC + lax.axis_index("c")
    iota = lax.broadcasted_iota(jnp.int32, (L,), 0)
    inf_v = jnp.full((L,), jnp.inf, jnp.float32)
    zero_i = jnp.zeros((L,), jnp.int32)

    for b in range(B):
        pltpu.sync_copy(xt_hbm.at[b], xv)
        pltpu.sync_copy(yt_hbm.at[b], yv)
        pltpu.sync_copy(zt_hbm.at[b], zv)
        pltpu.sync_copy(x2t_hbm.at[b], x2v)
        qoff0 = wid * (QPB * L)
        pltpu.sync_copy(qx_hbm.at[b, pl.ds(qoff0, QPB * L)], qxv)
        pltpu.sync_copy(qy_hbm.at[b, pl.ds(qoff0, QPB * L)], qyv)
        pltpu.sync_copy(qz_hbm.at[b, pl.ds(qoff0, QPB * L)], qzv)
        pltpu.sync_copy(c2_hbm.at[b, pl.ds(qoff0, QPB * L)], c2sv)

        def q_body(q4, _):
            q0 = q4 * QU
            qoffs = [(q0 + u) * L for u in range(QU)]
            qx_u = [qxv[pl.ds(qo, L)] for qo in qoffs]
            qy_u = [qyv[pl.ds(qo, L)] for qo in qoffs]
            qz_u = [qzv[pl.ds(qo, L)] for qo in qoffs]
            c2_u = [c2sv[pl.ds(qo, L)] for qo in qoffs]

            @plsc.parallel_loop(0, NSC, 1, unroll=2)
            def s_body(s):
                mv = [inf_v] * QU
                base = s * (L * L)
                for r in range(L):
                    off = base + r * L
                    xc = xv[pl.ds(off, L)]
                    yc = yv[pl.ds(off, L)]
                    zc = zv[pl.ds(off, L)]
                    x2c = x2v[pl.ds(off, L)]
                    for u in range(QU):
                        d = (c2_u[u] + ((qx_u[u] * xc + qy_u[u] * yc)
                                        + qz_u[u] * zc)) + x2c
                        mv[u] = jnp.minimum(mv[u], d)
                for u in range(QU):
                    cmv[pl.ds(u * NCH + s * L, L)] = mv[u]

            for u in range(QU):
                qx, qy, qz, c2, qoff = qx_u[u], qy_u[u], qz_u[u], c2_u[u], qoffs[u]

                def sm_body(s, carry):
                    sm1, sm2 = carry
                    c = cmv[pl.ds(u * NCH + s * L, L)]
                    return (jnp.minimum(sm1, c),
                            jnp.minimum(sm2, jnp.maximum(sm1, c)))

                sm1, sm2 = lax.fori_loop(0, NSC, sm_body, (inf_v, inf_v))
                s1d, _ = plsc.sort_key_val(sm1, zero_i)
                s2d, _ = plsc.sort_key_val(sm2, zero_i)
                low16 = jnp.minimum(s1d, lax.rev(s2d, (0,)))
                thr = jnp.max(low16)

                def cc_body(s, cursor):
                    c = cmv[pl.ds(u * NCH + s * L, L)]
                    msk = c <= thr
                    inc = jnp.where(msk, 1, 0).astype(jnp.int32)
                    pos = cursor + plsc.cumsum(inc) - 1
                    plsc.store_scatter(ccv, [pos], s * L + iota, mask=msk)
                    return cursor + plsc.all_reduce_population_count(msk)

                cursor = lax.fori_loop(0, NSC, cc_body, zero_i)
                n_cand = jnp.max(cursor)

                def pc_body(j, carry):
                    fd, fi = carry
                    cv = ccv[pl.ds((j >> 4) * L, L)]
                    cid = jnp.sum(jnp.where(iota == (j & (L - 1)), cv, 0))
                    pos = (cid >> 4) * (L * L) + (cid & (L - 1)) + iota * L
                    xc = plsc.load_gather(xv, [pos])
                    yc = plsc.load_gather(yv, [pos])
                    zc = plsc.load_gather(zv, [pos])
                    x2c = plsc.load_gather(x2v, [pos])
                    d = (c2 + ((qx * xc + qy * yc) + qz * zc)) + x2c
                    pid = cid * L + iota
                    return _merge_sorted(fd, fi, d, pid)

                fin_d, fin_i = lax.fori_loop(0, n_cand, pc_body,
                                             (inf_v, zero_i))
                odv[pl.ds(qoff, L)] = fin_d
                oiv[pl.ds(qoff, L)] = fin_i
            return 0

        lax.fori_loop(0, QPB // QU, q_body, 0)
        pltpu.sync_copy(odv, od_hbm.at[b, pl.ds(qoff0, QPB * L)])
        pltpu.sync_copy(oiv, oi_hbm.at[b, pl.ds(qoff0, QPB * L)])


_sc_knn = functools.partial(
    pl.kernel,
    out_type=[
        jax.ShapeDtypeStruct((B, M * L), jnp.float32),
        jax.ShapeDtypeStruct((B, M * L), jnp.int32),
    ],
    mesh=plsc.VectorSubcoreMesh(
        core_axis_name="c", subcore_axis_name="s",
        num_cores=NC, num_subcores=NS),
    compiler_params=pltpu.CompilerParams(needs_layout_passes=False),
    scratch_types=[
        pltpu.VMEM((N,), jnp.float32),
        pltpu.VMEM((N,), jnp.float32),
        pltpu.VMEM((N,), jnp.float32),
        pltpu.VMEM((N,), jnp.float32),
        pltpu.VMEM((QPB * L,), jnp.float32),
        pltpu.VMEM((QPB * L,), jnp.float32),
        pltpu.VMEM((QPB * L,), jnp.float32),
        pltpu.VMEM((QPB * L,), jnp.float32),
        pltpu.VMEM((QU * NCH,), jnp.float32),
        pltpu.VMEM((NCH,), jnp.int32),
        pltpu.VMEM((QPB * L,), jnp.float32),
        pltpu.VMEM((QPB * L,), jnp.int32),
    ],
)(_sc_body)


@jax.jit
def _knn(center_xyz, xyz):
    xyzb = lax.reduce_precision(xyz, exponent_bits=8, mantissa_bits=7)
    cb = lax.reduce_precision(center_xyz, exponent_bits=8, mantissa_bits=7)
    xyzb = -2.0 * xyzb
    xt3 = xyzb.reshape(B, NSC, L, L, 3).transpose(0, 1, 3, 2, 4)
    xt = xt3[..., 0].reshape(B, N)
    yt = xt3[..., 1].reshape(B, N)
    zt = xt3[..., 2].reshape(B, N)
    x2 = jnp.sum(xyz * xyz, axis=-1)
    x2t = x2.reshape(B, NSC, L, L).transpose(0, 1, 3, 2).reshape(B, N)
    c2 = jnp.sum(center_xyz * center_xyz, axis=-1)
    qs = jnp.broadcast_to(cb[:, :, None, :], (B, M, L, 3))
    qx = qs[..., 0].reshape(B, M * L)
    qy = qs[..., 1].reshape(B, M * L)
    qz = qs[..., 2].reshape(B, M * L)
    c2s = jnp.broadcast_to(c2[:, :, None], (B, M, L)).reshape(B, M * L)
    od, oi = _sc_knn(xt, yt, zt, x2t, qx, qy, qz, c2s)
    return od.reshape(B, M, K), oi.reshape(B, M, K)


def kernel(center_xyz, xyz, points):
    del points
    return tuple(_knn(center_xyz, xyz))


# speedup vs baseline: 1.1635x; 1.1635x over previous
import functools

import jax
import jax.numpy as jnp
from jax import lax
from jax.experimental import pallas as pl
from jax.experimental.pallas import tpu as pltpu
from jax.experimental.pallas import tpu_sc as plsc

K = 16
L = 16
B, M, N = 4, 2048, 8192
NC, NS = 2, 16
NW = NC * NS
NCH = N // L
NSC = NCH // L
QPB = M // NW
QU = 4


def _merge_sorted(fd, fi, d, pid):
    bd_s, bi_s = plsc.sort_key_val(d, pid)
    bd_r = lax.rev(bd_s, (0,))
    bi_r = lax.rev(bi_s, (0,))
    take = (fd < bd_r) | ((fd == bd_r) & (fi < bi_r))
    md = jnp.where(take, fd, bd_r)
    mi = jnp.where(take, fi, bi_r)
    sd, si = plsc.sort_key_val(md, mi)
    return sd, si


def _sc_body(xt_hbm, yt_hbm, zt_hbm, x2t_hbm, qx_hbm, qy_hbm, qz_hbm, c2_hbm,
             od_hbm, oi_hbm,
             xv, yv, zv, x2v, qxv, qyv, qzv, c2sv, cmv, odv, oiv):
    wid = lax.axis_index("s") * NC + lax.axis_index("c")
    iota = lax.broadcasted_iota(jnp.int32, (L,), 0)
    inf_v = jnp.full((L,), jnp.inf, jnp.float32)
    zero_i = jnp.zeros((L,), jnp.int32)

    for b in range(B):
        pltpu.sync_copy(xt_hbm.at[b], xv)
        pltpu.sync_copy(yt_hbm.at[b], yv)
        pltpu.sync_copy(zt_hbm.at[b], zv)
        pltpu.sync_copy(x2t_hbm.at[b], x2v)
        qoff0 = wid * (QPB * L)
        qrow = wid * QPB
        pltpu.sync_copy(qx_hbm.at[b, pl.ds(qrow, QPB)], qxv)
        pltpu.sync_copy(qy_hbm.at[b, pl.ds(qrow, QPB)], qyv)
        pltpu.sync_copy(qz_hbm.at[b, pl.ds(qrow, QPB)], qzv)
        pltpu.sync_copy(c2_hbm.at[b, pl.ds(qrow, QPB)], c2sv)

        def q_body(q4, _):
            q0 = q4 * QU
            qoffs = [(q0 + u) * L for u in range(QU)]
            qsel = [jnp.full((L,), q0 + u, jnp.int32) for u in range(QU)]
            qx_u = [plsc.load_gather(qxv, [qsel[u]]) for u in range(QU)]
            qy_u = [plsc.load_gather(qyv, [qsel[u]]) for u in range(QU)]
            qz_u = [plsc.load_gather(qzv, [qsel[u]]) for u in range(QU)]
            c2_u = [plsc.load_gather(c2sv, [qsel[u]]) for u in range(QU)]

            @plsc.parallel_loop(0, NSC, 1, unroll=2)
            def s_body(s):
                mv = [inf_v] * QU
                base = s * (L * L)
                for r in range(L):
                    off = base + r * L
                    xc = xv[pl.ds(off, L)]
                    yc = yv[pl.ds(off, L)]
                    zc = zv[pl.ds(off, L)]
                    x2c = x2v[pl.ds(off, L)]
                    for u in range(QU):
                        d = (c2_u[u] + ((qx_u[u] * xc + qy_u[u] * yc)
                                        + qz_u[u] * zc)) + x2c
                        mv[u] = jnp.minimum(mv[u], d)
                for u in range(QU):
                    cmv[pl.ds(u * NCH + s * L, L)] = mv[u]

            for u in range(QU):
                qx, qy, qz, c2, qoff = qx_u[u], qy_u[u], qz_u[u], c2_u[u], qoffs[u]

                def pb_body(s, carry):
                    rd, ri = carry
                    cm = cmv[pl.ds(u * NCH + s * L, L)]
                    return _merge_sorted(rd, ri, cm, s * L + iota)

                _, cand_i = lax.fori_loop(0, NSC, pb_body, (inf_v, zero_i))

                def pc_body(j, carry):
                    fd, fi = carry
                    cid = jnp.sum(jnp.where(iota == j, cand_i, 0))
                    pos = (cid >> 4) * (L * L) + (cid & (L - 1)) + iota * L
                    xc = plsc.load_gather(xv, [pos])
                    yc = plsc.load_gather(yv, [pos])
                    zc = plsc.load_gather(zv, [pos])
                    x2c = plsc.load_gather(x2v, [pos])
                    d = (c2 + ((qx * xc + qy * yc) + qz * zc)) + x2c
                    pid = cid * L + iota
                    return _merge_sorted(fd, fi, d, pid)

                fin_d, fin_i = lax.fori_loop(0, K, pc_body, (inf_v, zero_i))
                odv[pl.ds(qoff, L)] = fin_d
                oiv[pl.ds(qoff, L)] = fin_i
            return 0

        lax.fori_loop(0, QPB // QU, q_body, 0)
        pltpu.sync_copy(odv, od_hbm.at[b, pl.ds(qoff0, QPB * L)])
        pltpu.sync_copy(oiv, oi_hbm.at[b, pl.ds(qoff0, QPB * L)])


_sc_knn = functools.partial(
    pl.kernel,
    out_type=[
        jax.ShapeDtypeStruct((B, M * L), jnp.float32),
        jax.ShapeDtypeStruct((B, M * L), jnp.int32),
    ],
    mesh=plsc.VectorSubcoreMesh(
        core_axis_name="c", subcore_axis_name="s",
        num_cores=NC, num_subcores=NS),
    compiler_params=pltpu.CompilerParams(needs_layout_passes=False),
    scratch_types=[
        pltpu.VMEM((N,), jnp.float32),
        pltpu.VMEM((N,), jnp.float32),
        pltpu.VMEM((N,), jnp.float32),
        pltpu.VMEM((N,), jnp.float32),
        pltpu.VMEM((QPB,), jnp.float32),
        pltpu.VMEM((QPB,), jnp.float32),
        pltpu.VMEM((QPB,), jnp.float32),
        pltpu.VMEM((QPB,), jnp.float32),
        pltpu.VMEM((QU * NCH,), jnp.float32),
        pltpu.VMEM((QPB * L,), jnp.float32),
        pltpu.VMEM((QPB * L,), jnp.int32),
    ],
)(_sc_body)


@jax.jit
def _knn(center_xyz, xyz):
    xyzb = lax.reduce_precision(xyz, exponent_bits=8, mantissa_bits=7)
    cb = lax.reduce_precision(center_xyz, exponent_bits=8, mantissa_bits=7)
    xyzb = -2.0 * xyzb
    xt3 = xyzb.reshape(B, NSC, L, L, 3).transpose(0, 1, 3, 2, 4)
    xt = xt3[..., 0].reshape(B, N)
    yt = xt3[..., 1].reshape(B, N)
    zt = xt3[..., 2].reshape(B, N)
    x2 = jnp.sum(xyz * xyz, axis=-1)
    x2t = x2.reshape(B, NSC, L, L).transpose(0, 1, 3, 2).reshape(B, N)
    c2 = jnp.sum(center_xyz * center_xyz, axis=-1)
    qx = cb[..., 0]
    qy = cb[..., 1]
    qz = cb[..., 2]
    od, oi = _sc_knn(xt, yt, zt, x2t, qx, qy, qz, c2)
    return od.reshape(B, M, K), oi.reshape(B, M, K)


def kernel(center_xyz, xyz, points):
    del points
    return tuple(_knn(center_xyz, xyz))


# speedup vs baseline: 1.2228x; 1.0510x over previous
import functools

import jax
import jax.numpy as jnp
from jax import lax
from jax.experimental import pallas as pl
from jax.experimental.pallas import tpu as pltpu
from jax.experimental.pallas import tpu_sc as plsc

K = 16
L = 16
B, M, N = 4, 2048, 8192
NC, NS = 2, 16
NW = NC * NS
NCH = N // L
NSC = NCH // L
QPB = M // NW
QU = 4


def _merge_sorted(fd, fi, d, pid):
    bd_s, bi_s = plsc.sort_key_val(d, pid)
    bd_r = lax.rev(bd_s, (0,))
    bi_r = lax.rev(bi_s, (0,))
    take = (fd < bd_r) | ((fd == bd_r) & (fi < bi_r))
    md = jnp.where(take, fd, bd_r)
    mi = jnp.where(take, fi, bi_r)
    sd, si = plsc.sort_key_val(md, mi)
    return sd, si


def _sc_body(xt_hbm, yt_hbm, zt_hbm, x2t_hbm, qx_hbm, qy_hbm, qz_hbm, c2_hbm,
             od_hbm, oi_hbm,
             xv, yv, zv, x2v, qxv, qyv, qzv, c2sv, cmv, odv, oiv):
    wid = lax.axis_index("s") * NC + lax.axis_index("c")
    iota = lax.broadcasted_iota(jnp.int32, (L,), 0)
    inf_v = jnp.full((L,), jnp.inf, jnp.float32)
    zero_i = jnp.zeros((L,), jnp.int32)

    for b in range(B):
        pltpu.sync_copy(xt_hbm.at[b], xv)
        pltpu.sync_copy(yt_hbm.at[b], yv)
        pltpu.sync_copy(zt_hbm.at[b], zv)
        pltpu.sync_copy(x2t_hbm.at[b], x2v)
        qoff0 = wid * (QPB * L)
        qrow = wid * QPB
        pltpu.sync_copy(qx_hbm.at[b, pl.ds(qrow, QPB)], qxv)
        pltpu.sync_copy(qy_hbm.at[b, pl.ds(qrow, QPB)], qyv)
        pltpu.sync_copy(qz_hbm.at[b, pl.ds(qrow, QPB)], qzv)
        pltpu.sync_copy(c2_hbm.at[b, pl.ds(qrow, QPB)], c2sv)

        def q_body(q4, _):
            q0 = q4 * QU
            qoffs = [(q0 + u) * L for u in range(QU)]
            qsel = [jnp.full((L,), q0 + u, jnp.int32) for u in range(QU)]
            qx_u = [plsc.load_gather(qxv, [qsel[u]]) for u in range(QU)]
            qy_u = [plsc.load_gather(qyv, [qsel[u]]) for u in range(QU)]
            qz_u = [plsc.load_gather(qzv, [qsel[u]]) for u in range(QU)]
            c2_u = [plsc.load_gather(c2sv, [qsel[u]]) for u in range(QU)]

            @plsc.parallel_loop(0, NSC, 1, unroll=2)
            def s_body(s):
                mv = [inf_v] * QU
                base = s * (L * L)
                for r in range(L):
                    off = base + r * L
                    xc = xv[pl.ds(off, L)]
                    yc = yv[pl.ds(off, L)]
                    zc = zv[pl.ds(off, L)]
                    x2c = x2v[pl.ds(off, L)]
                    for u in range(QU):
                        d = ((qx_u[u] * xc + qy_u[u] * yc)
                             + qz_u[u] * zc) + x2c
                        mv[u] = jnp.minimum(mv[u], d)
                for u in range(QU):
                    cmv[pl.ds(u * NCH + s * L, L)] = mv[u]

            for u in range(QU):
                qx, qy, qz, c2, qoff = qx_u[u], qy_u[u], qz_u[u], c2_u[u], qoffs[u]

                def pb_body(s, carry):
                    rd, ri = carry
                    cm = cmv[pl.ds(u * NCH + s * L, L)]
                    return _merge_sorted(rd, ri, cm, s * L + iota)

                _, cand_i = lax.fori_loop(0, NSC, pb_body, (inf_v, zero_i))

                def pc_body(j, carry):
                    fd, fi = carry
                    cid = jnp.sum(jnp.where(iota == j, cand_i, 0))
                    pos = (cid >> 4) * (L * L) + (cid & (L - 1)) + iota * L
                    xc = plsc.load_gather(xv, [pos])
                    yc = plsc.load_gather(yv, [pos])
                    zc = plsc.load_gather(zv, [pos])
                    x2c = plsc.load_gather(x2v, [pos])
                    d = (c2 + ((qx * xc + qy * yc) + qz * zc)) + x2c
                    pid = cid * L + iota
                    return _merge_sorted(fd, fi, d, pid)

                fin_d, fin_i = lax.fori_loop(0, K, pc_body, (inf_v, zero_i))
                odv[pl.ds(qoff, L)] = fin_d
                oiv[pl.ds(qoff, L)] = fin_i
            return 0

        lax.fori_loop(0, QPB // QU, q_body, 0)
        pltpu.sync_copy(odv, od_hbm.at[b, pl.ds(qoff0, QPB * L)])
        pltpu.sync_copy(oiv, oi_hbm.at[b, pl.ds(qoff0, QPB * L)])


_sc_knn = functools.partial(
    pl.kernel,
    out_type=[
        jax.ShapeDtypeStruct((B, M * L), jnp.float32),
        jax.ShapeDtypeStruct((B, M * L), jnp.int32),
    ],
    mesh=plsc.VectorSubcoreMesh(
        core_axis_name="c", subcore_axis_name="s",
        num_cores=NC, num_subcores=NS),
    compiler_params=pltpu.CompilerParams(needs_layout_passes=False),
    scratch_types=[
        pltpu.VMEM((N,), jnp.float32),
        pltpu.VMEM((N,), jnp.float32),
        pltpu.VMEM((N,), jnp.float32),
        pltpu.VMEM((N,), jnp.float32),
        pltpu.VMEM((QPB,), jnp.float32),
        pltpu.VMEM((QPB,), jnp.float32),
        pltpu.VMEM((QPB,), jnp.float32),
        pltpu.VMEM((QPB,), jnp.float32),
        pltpu.VMEM((QU * NCH,), jnp.float32),
        pltpu.VMEM((QPB * L,), jnp.float32),
        pltpu.VMEM((QPB * L,), jnp.int32),
    ],
)(_sc_body)


@jax.jit
def _knn(center_xyz, xyz):
    xyzb = lax.reduce_precision(xyz, exponent_bits=8, mantissa_bits=7)
    cb = lax.reduce_precision(center_xyz, exponent_bits=8, mantissa_bits=7)
    xyzb = -2.0 * xyzb
    xt3 = xyzb.reshape(B, NSC, L, L, 3).transpose(0, 1, 3, 2, 4)
    xt = xt3[..., 0].reshape(B, N)
    yt = xt3[..., 1].reshape(B, N)
    zt = xt3[..., 2].reshape(B, N)
    x2 = jnp.sum(xyz * xyz, axis=-1)
    x2t = x2.reshape(B, NSC, L, L).transpose(0, 1, 3, 2).reshape(B, N)
    c2 = jnp.sum(center_xyz * center_xyz, axis=-1)
    qx = cb[..., 0]
    qy = cb[..., 1]
    qz = cb[..., 2]
    od, oi = _sc_knn(xt, yt, zt, x2t, qx, qy, qz, c2)
    return od.reshape(B, M, K), oi.reshape(B, M, K)


def kernel(center_xyz, xyz, points):
    del points
    return tuple(_knn(center_xyz, xyz))


# speedup vs baseline: 1.2272x; 1.0036x over previous
import functools

import jax
import jax.numpy as jnp
from jax import lax
from jax.experimental import pallas as pl
from jax.experimental.pallas import tpu as pltpu
from jax.experimental.pallas import tpu_sc as plsc

K = 16
L = 16
B, M, N = 4, 2048, 8192
NC, NS = 2, 16
NW = NC * NS
NCH = N // L
NSC = NCH // L
QPB = M // NW
QU = 4


def _merge_sorted(fd, fi, d, pid):
    bd_s, bi_s = plsc.sort_key_val(d, pid)
    bd_r = lax.rev(bd_s, (0,))
    bi_r = lax.rev(bi_s, (0,))
    take = (fd < bd_r) | ((fd == bd_r) & (fi < bi_r))
    md = jnp.where(take, fd, bd_r)
    mi = jnp.where(take, fi, bi_r)
    sd, si = plsc.sort_key_val(md, mi)
    return sd, si


def _sc_body(xt_hbm, yt_hbm, zt_hbm, x2t_hbm, qx_hbm, qy_hbm, qz_hbm, c2_hbm,
             od_hbm, oi_hbm,
             xv, yv, zv, x2v, qxv, qyv, qzv, c2sv, cmv, odv, oiv):
    wid = lax.axis_index("s") * NC + lax.axis_index("c")
    iota = lax.broadcasted_iota(jnp.int32, (L,), 0)
    inf_v = jnp.full((L,), jnp.inf, jnp.float32)
    zero_i = jnp.zeros((L,), jnp.int32)

    for b in range(B):
        pltpu.sync_copy(xt_hbm.at[b], xv)
        pltpu.sync_copy(yt_hbm.at[b], yv)
        pltpu.sync_copy(zt_hbm.at[b], zv)
        pltpu.sync_copy(x2t_hbm.at[b], x2v)
        qoff0 = wid * (QPB * L)
        qrow = wid * QPB
        pltpu.sync_copy(qx_hbm.at[b, pl.ds(qrow, QPB)], qxv)
        pltpu.sync_copy(qy_hbm.at[b, pl.ds(qrow, QPB)], qyv)
        pltpu.sync_copy(qz_hbm.at[b, pl.ds(qrow, QPB)], qzv)
        pltpu.sync_copy(c2_hbm.at[b, pl.ds(qrow, QPB)], c2sv)

        def q_body(q4, _):
            q0 = q4 * QU
            qoffs = [(q0 + u) * L for u in range(QU)]
            qsel = [jnp.full((L,), q0 + u, jnp.int32) for u in range(QU)]
            qx_u = [plsc.load_gather(qxv, [qsel[u]]) for u in range(QU)]
            qy_u = [plsc.load_gather(qyv, [qsel[u]]) for u in range(QU)]
            qz_u = [plsc.load_gather(qzv, [qsel[u]]) for u in range(QU)]
            c2_u = [plsc.load_gather(c2sv, [qsel[u]]) for u in range(QU)]

            @plsc.parallel_loop(0, NSC, 1, unroll=4)
            def s_body(s):
                mv = [inf_v] * QU
                base = s * (L * L)
                for r in range(L):
                    off = base + r * L
                    xc = xv[pl.ds(off, L)]
                    yc = yv[pl.ds(off, L)]
                    zc = zv[pl.ds(off, L)]
                    x2c = x2v[pl.ds(off, L)]
                    for u in range(QU):
                        d = ((qx_u[u] * xc + qy_u[u] * yc)
                             + qz_u[u] * zc) + x2c
                        mv[u] = jnp.minimum(mv[u], d)
                for u in range(QU):
                    cmv[pl.ds(u * NCH + s * L, L)] = mv[u]

            for u in range(QU):
                qx, qy, qz, c2, qoff = qx_u[u], qy_u[u], qz_u[u], c2_u[u], qoffs[u]

                def pb_body(s, carry):
                    rd, ri = carry
                    cm = cmv[pl.ds(u * NCH + s * L, L)]
                    return _merge_sorted(rd, ri, cm, s * L + iota)

                _, cand_i = lax.fori_loop(0, NSC, pb_body, (inf_v, zero_i))

                def pc_body(j, carry):
                    fd, fi = carry
                    cid = jnp.sum(jnp.where(iota == j, cand_i, 0))
                    pos = (cid >> 4) * (L * L) + (cid & (L - 1)) + iota * L
                    xc = plsc.load_gather(xv, [pos])
                    yc = plsc.load_gather(yv, [pos])
                    zc = plsc.load_gather(zv, [pos])
                    x2c = plsc.load_gather(x2v, [pos])
                    d = (c2 + ((qx * xc + qy * yc) + qz * zc)) + x2c
                    pid = cid * L + iota
                    return _merge_sorted(fd, fi, d, pid)

                fin_d, fin_i = lax.fori_loop(0, K, pc_body, (inf_v, zero_i))
                odv[pl.ds(qoff, L)] = fin_d
                oiv[pl.ds(qoff, L)] = fin_i
            return 0

        lax.fori_loop(0, QPB // QU, q_body, 0)
        pltpu.sync_copy(odv, od_hbm.at[b, pl.ds(qoff0, QPB * L)])
        pltpu.sync_copy(oiv, oi_hbm.at[b, pl.ds(qoff0, QPB * L)])


_sc_knn = functools.partial(
    pl.kernel,
    out_type=[
        jax.ShapeDtypeStruct((B, M * L), jnp.float32),
        jax.ShapeDtypeStruct((B, M * L), jnp.int32),
    ],
    mesh=plsc.VectorSubcoreMesh(
        core_axis_name="c", subcore_axis_name="s",
        num_cores=NC, num_subcores=NS),
    compiler_params=pltpu.CompilerParams(needs_layout_passes=False),
    scratch_types=[
        pltpu.VMEM((N,), jnp.float32),
        pltpu.VMEM((N,), jnp.float32),
        pltpu.VMEM((N,), jnp.float32),
        pltpu.VMEM((N,), jnp.float32),
        pltpu.VMEM((QPB,), jnp.float32),
        pltpu.VMEM((QPB,), jnp.float32),
        pltpu.VMEM((QPB,), jnp.float32),
        pltpu.VMEM((QPB,), jnp.float32),
        pltpu.VMEM((QU * NCH,), jnp.float32),
        pltpu.VMEM((QPB * L,), jnp.float32),
        pltpu.VMEM((QPB * L,), jnp.int32),
    ],
)(_sc_body)


@jax.jit
def _knn(center_xyz, xyz):
    xyzb = lax.reduce_precision(xyz, exponent_bits=8, mantissa_bits=7)
    cb = lax.reduce_precision(center_xyz, exponent_bits=8, mantissa_bits=7)
    xyzb = -2.0 * xyzb
    xt3 = xyzb.reshape(B, NSC, L, L, 3).transpose(0, 1, 3, 2, 4)
    xt = xt3[..., 0].reshape(B, N)
    yt = xt3[..., 1].reshape(B, N)
    zt = xt3[..., 2].reshape(B, N)
    x2 = jnp.sum(xyz * xyz, axis=-1)
    x2t = x2.reshape(B, NSC, L, L).transpose(0, 1, 3, 2).reshape(B, N)
    c2 = jnp.sum(center_xyz * center_xyz, axis=-1)
    qx = cb[..., 0]
    qy = cb[..., 1]
    qz = cb[..., 2]
    od, oi = _sc_knn(xt, yt, zt, x2t, qx, qy, qz, c2)
    return od.reshape(B, M, K), oi.reshape(B, M, K)


def kernel(center_xyz, xyz, points):
    del points
    return tuple(_knn(center_xyz, xyz))
